# baseline (speedup 1.0000x reference)
import functools

import jax
import jax.numpy as jnp
from jax import lax
from jax.experimental import pallas as pl
from jax.experimental.pallas import tpu as pltpu
from jax.experimental.pallas import tpu_sc as plsc

_B, _N, _F = 4096, 32, 256
_NC, _NS, _L = 2, 16, 16
_NW = _NC * _NS
_RPW = _B // _NW

_mesh = plsc.VectorSubcoreMesh(core_axis_name="c", subcore_axis_name="s")


@functools.partial(
    pl.kernel,
    mesh=_mesh,
    out_type=jax.ShapeDtypeStruct((_B, 2), jnp.float32),
    scratch_types=[
        pltpu.VMEM((2, _RPW // 2), jnp.int32),
        pltpu.VMEM((2, _RPW // 2, _F), jnp.float32),
        pltpu.VMEM((_RPW, 2), jnp.float32),
        pltpu.SemaphoreType.DMA,
        pltpu.SemaphoreType.DMA,
    ],
    compiler_params=pltpu.CompilerParams(
        needs_layout_passes=False, skip_device_barrier=True
    ),
)
def _person_rule_sc(x_hbm, out_hbm, idx_v, rows_v, y_v, sem0, sem1):
    wid = lax.axis_index("s") * _NC + lax.axis_index("c")
    base = wid * _RPW
    half = _RPW // 2
    iota = lax.broadcasted_iota(jnp.int32, (_L,), 0)
    sems = (sem0, sem1)
    for h in range(2):
        for i in range(half // _L):
            idx_v[h, pl.ds(i * _L, _L)] = (
                base + h * half + i * _L + iota
            ) * _N + 2
        pltpu.async_copy(x_hbm.at[idx_v.at[h]], rows_v.at[h], sems[h])
    zeros = jnp.zeros((_L,), jnp.int32)
    ones = jnp.ones((_L,), jnp.int32)
    for h in range(2):
        pltpu.make_async_copy(x_hbm.at[idx_v.at[h]], rows_v.at[h], sems[h]).wait()
        for i in range(half // _L):
            ridx = iota + (i * _L)
            v0 = plsc.load_gather(rows_v.at[h], [ridx, zeros])
            v1 = plsc.load_gather(rows_v.at[h], [ridx, ones])
            t0 = jnp.where(v0 > 0, 1.0, v0)
            t1 = jnp.where(v1 > 0, 1.0, v1)
            zb = t0 + t1
            y0 = jnp.where(zb == 0, 100.0, -100.0)
            y1 = jnp.where(zb > 0, 100.0, -100.0)
            plsc.store_scatter(y_v, [ridx + h * half, zeros], y0)
            plsc.store_scatter(y_v, [ridx + h * half, ones], y1)
    pltpu.sync_copy(y_v, out_hbm.at[pl.ds(base, _RPW)])


def kernel(x, adj_mat):
    del adj_mat
    return _person_rule_sc(x.reshape(_B * _N, _F))
